# Initial kernel scaffold; baseline (speedup 1.0000x reference)
import jax
import jax.numpy as jnp
from jax.experimental import pallas as pl


def kernel(x, W, att_src, att_dst, bias):
    raise NotImplementedError("write your pallas kernel here")


# speedup vs baseline: 4673.4099x; 4673.4099x over previous
import jax
import jax.numpy as jnp
from jax.experimental import pallas as pl


def _gat_body(x_ref, w_ref, att_ref, bias_ref, out_ref):
    p = x_ref.shape[0]
    h = jnp.dot(x_ref[...], w_ref[...], preferred_element_type=jnp.float32)
    sd = jnp.dot(h, att_ref[...], preferred_element_type=jnp.float32)
    s = sd[:, 0]
    d = sd[:, 1]
    e = d[:, None] + s[None, :]
    e = jnp.where(e >= 0, e, 0.2 * e)
    row = jax.lax.broadcasted_iota(jnp.int32, (p, p), 0)
    col = jax.lax.broadcasted_iota(jnp.int32, (p, p), 1)
    e = jnp.where(col <= row, e, -jnp.inf)
    m = jnp.max(e, axis=1, keepdims=True)
    ex = jnp.exp(e - m)
    denom = jnp.sum(ex, axis=1, keepdims=True)
    a = ex / denom
    out = jnp.dot(a, h, preferred_element_type=jnp.float32) + bias_ref[...]
    out_ref[...] = jnp.maximum(out, 0.0)


def kernel(x, W, att_src, att_dst, bias):
    n, din = x.shape
    dout = W.shape[1]
    pad = (-n) % 128
    p = n + pad
    x_p = jnp.pad(x, ((0, pad), (0, 0)))
    att = jnp.stack([att_src, att_dst], axis=1)
    out = pl.pallas_call(
        _gat_body,
        out_shape=jax.ShapeDtypeStruct((p, dout), jnp.float32),
    )(x_p, W, att, bias[None, :])
    return out[:n]


# speedup vs baseline: 4775.3017x; 1.0218x over previous
import jax
import jax.numpy as jnp
from jax.experimental import pallas as pl


def _gat_body(x_ref, w_ref, att_ref, bias_ref, out_ref):
    p = x_ref.shape[0]
    dout = w_ref.shape[1]
    h = jnp.dot(x_ref[...], w_ref[...], preferred_element_type=jnp.float32)
    sd = jnp.dot(h, att_ref[...], preferred_element_type=jnp.float32)
    s = sd[:, 0]
    d = sd[:, 1]
    e = d[:, None] + s[None, :]
    e = jnp.where(e >= 0, e, 0.2 * e)
    row = jax.lax.broadcasted_iota(jnp.int32, (p, p), 0)
    col = jax.lax.broadcasted_iota(jnp.int32, (p, p), 1)
    e = jnp.where(col <= row, e, -jnp.inf)
    m = jnp.max(e, axis=1, keepdims=True)
    ex = jnp.exp(e - m)
    h1 = jnp.concatenate([h, jnp.ones((p, 1), jnp.float32)], axis=1)
    acc = jnp.dot(ex, h1, preferred_element_type=jnp.float32)
    out = acc[:, :dout] / acc[:, dout:dout + 1] + bias_ref[...]
    out_ref[...] = jnp.maximum(out, 0.0)


def kernel(x, W, att_src, att_dst, bias):
    n, din = x.shape
    dout = W.shape[1]
    pad = (-n) % 128
    p = n + pad
    x_p = jnp.pad(x, ((0, pad), (0, 0)))
    att = jnp.stack([att_src, att_dst], axis=1)
    out = pl.pallas_call(
        _gat_body,
        out_shape=jax.ShapeDtypeStruct((p, dout), jnp.float32),
    )(x_p, W, att, bias[None, :])
    return out[:n]


# speedup vs baseline: 6034.4919x; 1.2637x over previous
import jax
import jax.numpy as jnp
from jax.experimental import pallas as pl


def _gat_body(x_ref, w_ref, att_s_ref, att_d_ref, bias_ref, out_ref):
    p = x_ref.shape[0]
    dout = w_ref.shape[1]
    h = jnp.dot(x_ref[...], w_ref[...], preferred_element_type=jnp.float32)
    s = jnp.sum(h * att_s_ref[...], axis=1)
    d = jnp.sum(h * att_d_ref[...], axis=1)
    e = d[:, None] + s[None, :]
    e = jnp.where(e >= 0, e, 0.2 * e)
    row = jax.lax.broadcasted_iota(jnp.int32, (p, p), 0)
    col = jax.lax.broadcasted_iota(jnp.int32, (p, p), 1)
    e = jnp.where(col <= row, e, -jnp.inf)
    m = jnp.max(e, axis=1, keepdims=True)
    ex = jnp.exp(e - m)
    h1 = jnp.concatenate([h, jnp.ones((p, 1), jnp.float32)], axis=1)
    acc = jnp.dot(ex, h1, preferred_element_type=jnp.float32)
    out = acc[:, :dout] / acc[:, dout:dout + 1] + bias_ref[...]
    out_ref[...] = jnp.maximum(out, 0.0)


def kernel(x, W, att_src, att_dst, bias):
    n, _ = x.shape
    dout = W.shape[1]
    return pl.pallas_call(
        _gat_body,
        out_shape=jax.ShapeDtypeStruct((n, dout), jnp.float32),
    )(x, W, att_src[None, :], att_dst[None, :], bias[None, :])


# speedup vs baseline: 6247.3140x; 1.0353x over previous
import jax
import jax.numpy as jnp
from jax.experimental import pallas as pl

_ROW_BLOCK = 512


def _gat_body(x_ref, w_ref, att_s_ref, att_d_ref, bias_ref, out_ref):
    p = x_ref.shape[0]
    dout = w_ref.shape[1]
    h = jnp.dot(x_ref[...], w_ref[...], preferred_element_type=jnp.float32)
    s = jnp.sum(h * att_s_ref[...], axis=1)
    d = jnp.sum(h * att_d_ref[...], axis=1)
    h1 = jnp.concatenate([h, jnp.ones((p, 1), jnp.float32)], axis=1)
    bias = bias_ref[...]

    def leaky(v):
        return jnp.where(v >= 0, v, 0.2 * v)

    for r0 in range(0, p, _ROW_BLOCK):
        rn = min(_ROW_BLOCK, p - r0)
        db = d[r0:r0 + rn][:, None]
        ed = leaky(db + s[None, r0:r0 + rn])
        row = jax.lax.broadcasted_iota(jnp.int32, (rn, rn), 0)
        col = jax.lax.broadcasted_iota(jnp.int32, (rn, rn), 1)
        ed = jnp.where(col <= row, ed, -jnp.inf)
        md = jnp.max(ed, axis=1, keepdims=True)
        if r0 > 0:
            el = leaky(db + s[None, :r0])
            m = jnp.maximum(jnp.max(el, axis=1, keepdims=True), md)
            acc = (
                jnp.dot(jnp.exp(el - m), h1[:r0],
                        preferred_element_type=jnp.float32)
                + jnp.dot(jnp.exp(ed - m), h1[r0:r0 + rn],
                          preferred_element_type=jnp.float32)
            )
        else:
            acc = jnp.dot(jnp.exp(ed - md), h1[:rn],
                          preferred_element_type=jnp.float32)
        out = acc[:, :dout] / acc[:, dout:dout + 1] + bias
        out_ref[r0:r0 + rn, :] = jnp.maximum(out, 0.0)


def kernel(x, W, att_src, att_dst, bias):
    n, _ = x.shape
    dout = W.shape[1]
    return pl.pallas_call(
        _gat_body,
        out_shape=jax.ShapeDtypeStruct((n, dout), jnp.float32),
    )(x, W, att_src[None, :], att_dst[None, :], bias[None, :])
